# baseline (speedup 1.0000x reference)
import functools

import jax
import jax.numpy as jnp
from jax import lax
from jax.experimental import pallas as pl
from jax.experimental.pallas import tpu as pltpu
from jax.experimental.pallas import tpu_sc as plsc

VOCAB = 100000
EMB = 128
BATCH = 4096
SEQ = 200
OUT_PAD = 32

_TC_BLK = 4000
_PREC = lax.Precision.HIGHEST

_COL_PERM = [(j // 2) + 16 * (j % 2) for j in range(OUT_PAD)]


def _tc_w12_body(w1_ref, w2p_ref, w2i_ref, b1_ref, b2_ref,
                 w12_ref, bias_ref):
    w12_ref[...] = jnp.dot(w1_ref[...], w2i_ref[...],
                           preferred_element_type=jnp.float32,
                           precision=_PREC)
    bias_ref[...] = jnp.dot(b1_ref[...], w2p_ref[...],
                            preferred_element_type=jnp.float32,
                            precision=_PREC) + b2_ref[...]


def _tc_fold_body(tbl_ref, w12_ref, t2_ref):
    t2_ref[...] = jnp.dot(tbl_ref[...], w12_ref[...],
                          preferred_element_type=jnp.float32,
                          precision=_PREC).astype(jnp.bfloat16)


def _fold_table(table, W1, W2p, W2i, b1r, b2r):
    w12, bias = pl.pallas_call(
        _tc_w12_body,
        out_shape=[
            jax.ShapeDtypeStruct((EMB, OUT_PAD), jnp.float32),
            jax.ShapeDtypeStruct((1, OUT_PAD), jnp.float32),
        ],
    )(W1, W2p, W2i, b1r, b2r)
    t2 = pl.pallas_call(
        _tc_fold_body,
        grid=(VOCAB // _TC_BLK,),
        in_specs=[
            pl.BlockSpec((_TC_BLK, EMB), lambda i: (i, 0)),
            pl.BlockSpec((EMB, OUT_PAD), lambda i: (0, 0)),
        ],
        out_specs=pl.BlockSpec((_TC_BLK, OUT_PAD), lambda i: (i, 0)),
        out_shape=jax.ShapeDtypeStruct((VOCAB, OUT_PAD), jnp.bfloat16),
    )(table, w12)
    return t2, bias


_INFO = plsc.get_sparse_core_info()
_NC, _NS, _L = _INFO.num_cores, _INFO.num_subcores, _INFO.num_lanes
_NW = _NC * _NS
_B_PER_W = BATCH // _NW
_HALF = SEQ // 2
_HR_PER_W = 2 * _B_PER_W
_CHUNK_ROWS = 4
_CHUNK_HR = 2 * _CHUNK_ROWS
_N_CHUNKS = _B_PER_W // _CHUNK_ROWS
_BUF_ROWS = _CHUNK_HR * _HALF


def _sc_pool_body(xr_hbm, t2_hbm, bias_hbm, out_hbm,
                  idx_v, buf_a, buf_b, out_v, bias_v, sem_a, sem_b):
    wid = lax.axis_index("s") * _NC + lax.axis_index("c")
    pltpu.sync_copy(xr_hbm.at[pl.ds(wid * _HR_PER_W, _HR_PER_W)], idx_v)
    pltpu.sync_copy(bias_hbm, bias_v)
    bias0 = bias_v[pl.ds(0, _L)]
    bias1 = bias_v[pl.ds(_L, _L)]

    def fire(c, buf, sem):
        for k in range(_CHUNK_HR):
            pltpu.async_copy(t2_hbm.at[idx_v.at[_CHUNK_HR * c + k]],
                             buf.at[pl.ds(k * _HALF, _HALF)], sem)

    def drain(buf, sem):
        pltpu.make_async_copy(
            t2_hbm.at[pl.ds(0, _BUF_ROWS)], buf, sem).wait()

    def accum_chunk(c, buf):
        for r in range(_CHUNK_ROWS):
            base = r * SEQ

            def jbody(ji, accs, base=base):
                new = list(accs)
                o = base + ji * 8
                for jj in range(8):
                    row = buf[o + jj, pl.ds(0, OUT_PAD)]
                    lo, hi = plsc.unpack(
                        row, format=plsc.PackFormat.INTERLEAVED)
                    a = jj % 4
                    new[a] = new[a] + lo
                    new[4 + a] = new[4 + a] + hi
                return tuple(new)

            z = jnp.zeros((_L,), jnp.float32)
            accs = lax.fori_loop(0, SEQ // 8, jbody, (z,) * 8)
            a0 = (accs[0] + accs[1]) + (accs[2] + accs[3])
            a1 = (accs[4] + accs[5]) + (accs[6] + accs[7])
            rg = c * _CHUNK_ROWS + r
            out_v[rg, pl.ds(0, _L)] = a0 + bias0
            out_v[rg, pl.ds(_L, _L)] = a1 + bias1

    fire(0, buf_a, sem_a)

    def body(i, carry):
        for b in range(2):
            cbuf, csem = (buf_a, sem_a) if b == 0 else (buf_b, sem_b)
            nbuf, nsem = (buf_b, sem_b) if b == 0 else (buf_a, sem_a)
            c = 2 * i + b

            @pl.when(c < _N_CHUNKS - 1)
            def _(c=c, nbuf=nbuf, nsem=nsem):
                fire(c + 1, nbuf, nsem)

            drain(cbuf, csem)
            accum_chunk(c, cbuf)
        return carry

    lax.fori_loop(0, _N_CHUNKS // 2, body, 0)
    pltpu.sync_copy(out_v, out_hbm.at[pl.ds(wid * _B_PER_W, _B_PER_W)])


_sc_pool = functools.partial(
    pl.kernel,
    out_type=jax.ShapeDtypeStruct((BATCH, OUT_PAD), jnp.float32),
    mesh=plsc.VectorSubcoreMesh(core_axis_name="c", subcore_axis_name="s"),
    compiler_params=pltpu.CompilerParams(use_tc_tiling_on_sc=False,
                                         needs_layout_passes=False),
    scratch_types=[
        pltpu.VMEM((_HR_PER_W, _HALF), jnp.int32),
        pltpu.VMEM((_BUF_ROWS, OUT_PAD), jnp.bfloat16),
        pltpu.VMEM((_BUF_ROWS, OUT_PAD), jnp.bfloat16),
        pltpu.VMEM((_B_PER_W, OUT_PAD), jnp.float32),
        pltpu.VMEM((OUT_PAD,), jnp.float32),
        pltpu.SemaphoreType.DMA,
        pltpu.SemaphoreType.DMA,
    ],
)(_sc_pool_body)


def kernel(x, table, W1, b1, W2, b2):
    W2p = jnp.pad(W2, ((0, 0), (0, OUT_PAD - W2.shape[1])))
    W2i = W2p[:, jnp.array(_COL_PERM, dtype=jnp.int32)]
    b2r = jnp.pad(b2, (0, OUT_PAD - b2.shape[0])).reshape(1, OUT_PAD)
    b1r = b1.reshape(1, EMB)
    t2, bias = _fold_table(table, W1, W2p, W2i, b1r, b2r)
    xr = x.astype(jnp.int32).reshape(BATCH * 2, _HALF)
    pooled = _sc_pool(xr, t2, bias.reshape(OUT_PAD))
    return pooled[:, : b2.shape[0]]


# speedup vs baseline: 1.1726x; 1.1726x over previous
import functools

import jax
import jax.numpy as jnp
from jax import lax
from jax.experimental import pallas as pl
from jax.experimental.pallas import tpu as pltpu
from jax.experimental.pallas import tpu_sc as plsc

VOCAB = 100000
EMB = 128
BATCH = 4096
SEQ = 200
OUT = 19
OUT_PAD = 32

_TC_BLK = 10000
_PREC = lax.Precision.HIGHEST

_COL_PERM = [(j // 2) + 16 * (j % 2) for j in range(OUT_PAD)]


def _tc_fold_body(tbl_ref, w1_ref, w2_ref, b1_ref, b2_ref,
                  t2_ref, bias_ref):
    w2p = jnp.pad(w2_ref[...], ((0, 0), (0, OUT_PAD - OUT)))
    b2p = jnp.pad(b2_ref[...], ((0, 0), (0, OUT_PAD - OUT)))
    rowid = lax.broadcasted_iota(jnp.int32, (OUT_PAD, OUT_PAD), 0)
    colid = lax.broadcasted_iota(jnp.int32, (OUT_PAD, OUT_PAD), 1)
    perm = colid // 2 + (colid % 2) * _L
    pmat = (rowid == perm).astype(jnp.float32)
    w2i = jnp.dot(w2p, pmat, preferred_element_type=jnp.float32,
                  precision=_PREC)
    w12 = jnp.dot(w1_ref[...], w2i, preferred_element_type=jnp.float32,
                  precision=_PREC)
    t2_ref[...] = jnp.dot(
        tbl_ref[...].astype(jnp.bfloat16), w12.astype(jnp.bfloat16),
        preferred_element_type=jnp.float32).astype(jnp.bfloat16)
    bias_ref[...] = jnp.dot(b1_ref[...], w2p,
                            preferred_element_type=jnp.float32,
                            precision=_PREC) + b2p


def _fold_table(table, W1, W2, b1r, b2r):
    return pl.pallas_call(
        _tc_fold_body,
        grid=(VOCAB // _TC_BLK,),
        in_specs=[
            pl.BlockSpec((_TC_BLK, EMB), lambda i: (i, 0)),
            pl.BlockSpec((EMB, EMB), lambda i: (0, 0)),
            pl.BlockSpec((EMB, OUT), lambda i: (0, 0)),
            pl.BlockSpec((1, EMB), lambda i: (0, 0)),
            pl.BlockSpec((1, OUT), lambda i: (0, 0)),
        ],
        out_specs=[
            pl.BlockSpec((_TC_BLK, OUT_PAD), lambda i: (i, 0)),
            pl.BlockSpec((1, OUT_PAD), lambda i: (0, 0)),
        ],
        out_shape=[
            jax.ShapeDtypeStruct((VOCAB, OUT_PAD), jnp.bfloat16),
            jax.ShapeDtypeStruct((1, OUT_PAD), jnp.float32),
        ],
    )(table, W1, W2, b1r, b2r)


_INFO = plsc.get_sparse_core_info()
_NC, _NS, _L = _INFO.num_cores, _INFO.num_subcores, _INFO.num_lanes
_NW = _NC * _NS
_B_PER_W = BATCH // _NW
_HALF = SEQ // 2
_HR_PER_W = 2 * _B_PER_W
_CHUNK_ROWS = 8
_CHUNK_HR = 2 * _CHUNK_ROWS
_N_CHUNKS = _B_PER_W // _CHUNK_ROWS
_BUF_ROWS = _CHUNK_HR * _HALF


def _sc_pool_body(xr_hbm, t2_hbm, bias_hbm, out_hbm,
                  idx_v, buf_a, buf_b, out_v, bias_v, sem_a, sem_b):
    wid = lax.axis_index("s") * _NC + lax.axis_index("c")
    pltpu.sync_copy(xr_hbm.at[pl.ds(wid * _HR_PER_W, _HR_PER_W)], idx_v)
    pltpu.sync_copy(bias_hbm, bias_v)
    bias0 = bias_v[pl.ds(0, _L)]
    bias1 = bias_v[pl.ds(_L, _L)]

    def fire(c, buf, sem):
        for k in range(_CHUNK_HR):
            pltpu.async_copy(t2_hbm.at[idx_v.at[_CHUNK_HR * c + k]],
                             buf.at[pl.ds(k * _HALF, _HALF)], sem)

    def drain(buf, sem):
        pltpu.make_async_copy(
            t2_hbm.at[pl.ds(0, _BUF_ROWS)], buf, sem).wait()

    def accum_chunk(c, buf):
        for r in range(_CHUNK_ROWS):
            base = r * SEQ

            def jbody(ji, accs, base=base):
                new = list(accs)
                o = base + ji * 8
                for jj in range(8):
                    row = buf[o + jj, pl.ds(0, OUT_PAD)]
                    lo, hi = plsc.unpack(
                        row, format=plsc.PackFormat.INTERLEAVED)
                    a = jj % 4
                    new[a] = new[a] + lo
                    new[4 + a] = new[4 + a] + hi
                return tuple(new)

            z = jnp.zeros((_L,), jnp.float32)
            accs = lax.fori_loop(0, SEQ // 8, jbody, (z,) * 8)
            a0 = (accs[0] + accs[1]) + (accs[2] + accs[3])
            a1 = (accs[4] + accs[5]) + (accs[6] + accs[7])
            rg = c * _CHUNK_ROWS + r
            out_v[rg, pl.ds(0, _L)] = a0 + bias0
            out_v[rg, pl.ds(_L, _L)] = a1 + bias1

    fire(0, buf_a, sem_a)

    def body(i, carry):
        for b in range(2):
            cbuf, csem = (buf_a, sem_a) if b == 0 else (buf_b, sem_b)
            nbuf, nsem = (buf_b, sem_b) if b == 0 else (buf_a, sem_a)
            c = 2 * i + b

            @pl.when(c < _N_CHUNKS - 1)
            def _(c=c, nbuf=nbuf, nsem=nsem):
                fire(c + 1, nbuf, nsem)

            drain(cbuf, csem)
            accum_chunk(c, cbuf)
        return carry

    lax.fori_loop(0, _N_CHUNKS // 2, body, 0)
    pltpu.sync_copy(out_v, out_hbm.at[pl.ds(wid * _B_PER_W, _B_PER_W)])


_sc_pool = functools.partial(
    pl.kernel,
    out_type=jax.ShapeDtypeStruct((BATCH, OUT_PAD), jnp.float32),
    mesh=plsc.VectorSubcoreMesh(core_axis_name="c", subcore_axis_name="s"),
    compiler_params=pltpu.CompilerParams(use_tc_tiling_on_sc=False,
                                         needs_layout_passes=False),
    scratch_types=[
        pltpu.VMEM((_HR_PER_W, _HALF), jnp.int32),
        pltpu.VMEM((_BUF_ROWS, OUT_PAD), jnp.bfloat16),
        pltpu.VMEM((_BUF_ROWS, OUT_PAD), jnp.bfloat16),
        pltpu.VMEM((_B_PER_W, OUT_PAD), jnp.float32),
        pltpu.VMEM((OUT_PAD,), jnp.float32),
        pltpu.SemaphoreType.DMA,
        pltpu.SemaphoreType.DMA,
    ],
)(_sc_pool_body)


def kernel(x, table, W1, b1, W2, b2):
    b1r = b1.reshape(1, EMB)
    b2r = b2.reshape(1, OUT)
    t2, bias = _fold_table(table, W1, W2, b1r, b2r)
    xr = x.astype(jnp.int32).reshape(BATCH * 2, _HALF)
    pooled = _sc_pool(xr, t2, bias.reshape(OUT_PAD))
    return pooled[:, :OUT]
